# baseline (speedup 1.0000x reference)
import functools

import jax
import jax.numpy as jnp
from jax.experimental import pallas as pl
from jax.experimental.pallas import tpu as pltpu

B = 16
TOTAL = 16384
D_FEAT = 256
D_MODEL = 1024
AUX_OUT = 1
TILE = 2048
NUM_TILES = TOTAL // TILE


def _fused_kernel(ent_ref, bi_ref, we_ref, be_ref, wb_ref, bb_ref, wa_ref,
                  ba_ref, out_ref, acc_ref, cnt_ref):
    i = pl.program_id(0)

    @pl.when(i == 0)
    def _init():
        acc_ref[...] = jnp.zeros_like(acc_ref)
        cnt_ref[...] = jnp.zeros_like(cnt_ref)

    x = jnp.dot(ent_ref[...], we_ref[...], preferred_element_type=jnp.float32)
    x = jnp.maximum(x + be_ref[...], 0.0)
    h = jnp.dot(x.astype(jnp.bfloat16), wb_ref[...],
                preferred_element_type=jnp.float32)
    h = jnp.maximum(h + bb_ref[...], 0.0)

    bi = bi_ref[0, :]
    oh_t = (jax.lax.broadcasted_iota(jnp.int32, (B, TILE), 0)
            == bi[None, :]).astype(jnp.bfloat16)
    acc_ref[...] += jnp.dot(oh_t, h.astype(jnp.bfloat16),
                            preferred_element_type=jnp.float32)
    cnt_ref[...] += jnp.broadcast_to(
        jnp.sum(oh_t.astype(jnp.float32), axis=1, keepdims=True), (B, 128))

    @pl.when(i == NUM_TILES - 1)
    def _finalize():
        counts = cnt_ref[:, 0:1]
        pooled = acc_ref[...] / jnp.maximum(counts, 1.0)
        aux = jnp.dot(pooled, wa_ref[...], preferred_element_type=jnp.float32)
        out_ref[...] = aux + ba_ref[...]


@functools.partial(jax.jit, static_argnames=())
def kernel(entities, batch_index, W_embed, b_embed, W_bb, b_bb, W_aux, b_aux):
    bi = batch_index.astype(jnp.int32).reshape(NUM_TILES, 1, TILE)
    entities = entities.astype(jnp.bfloat16)
    W_embed = W_embed.astype(jnp.bfloat16)
    W_bb = W_bb.astype(jnp.bfloat16)
    grid = (NUM_TILES,)
    out = pl.pallas_call(
        _fused_kernel,
        grid=grid,
        in_specs=[
            pl.BlockSpec((TILE, D_FEAT), lambda i: (i, 0)),
            pl.BlockSpec((None, 1, TILE), lambda i: (i, 0, 0)),
            pl.BlockSpec((D_FEAT, D_MODEL), lambda i: (0, 0)),
            pl.BlockSpec((1, D_MODEL), lambda i: (0, 0)),
            pl.BlockSpec((D_MODEL, D_MODEL), lambda i: (0, 0)),
            pl.BlockSpec((1, D_MODEL), lambda i: (0, 0)),
            pl.BlockSpec((D_MODEL, AUX_OUT), lambda i: (0, 0)),
            pl.BlockSpec((1, AUX_OUT), lambda i: (0, 0)),
        ],
        out_specs=pl.BlockSpec((B, AUX_OUT), lambda i: (0, 0)),
        out_shape=jax.ShapeDtypeStruct((B, AUX_OUT), jnp.float32),
        scratch_shapes=[
            pltpu.VMEM((B, D_MODEL), jnp.float32),
            pltpu.VMEM((B, 128), jnp.float32),
        ],
    )(entities, bi, W_embed, b_embed.reshape(1, D_MODEL), W_bb,
      b_bb.reshape(1, D_MODEL), W_aux, b_aux.reshape(1, AUX_OUT))
    return out


# speedup vs baseline: 1.2540x; 1.2540x over previous
import functools

import jax
import jax.numpy as jnp
from jax.experimental import pallas as pl
from jax.experimental.pallas import tpu as pltpu

B = 16
TOTAL = 16384
D_FEAT = 256
D_MODEL = 1024
AUX_OUT = 1
TILE = 2048
NUM_TILES = TOTAL // TILE


def _fused_kernel(ent_ref, bi_ref, we_ref, be_ref, wb_ref, bb_ref, wa_ref,
                  ba_ref, out_ref, acc_ref, cnt_ref):
    i = pl.program_id(0)

    @pl.when(i == 0)
    def _init():
        acc_ref[...] = jnp.zeros_like(acc_ref)
        cnt_ref[...] = jnp.zeros_like(cnt_ref)

    x = jnp.dot(ent_ref[...], we_ref[...], preferred_element_type=jnp.float32)
    x = jnp.maximum(x + be_ref[...], 0.0)
    h = jnp.dot(x, wb_ref[...], preferred_element_type=jnp.float32)
    h = jnp.maximum(h + bb_ref[...], 0.0)

    bi = bi_ref[0, :]
    oh_t = (jax.lax.broadcasted_iota(jnp.int32, (B, TILE), 0)
            == bi[None, :]).astype(jnp.float32)
    acc_ref[...] += jnp.dot(oh_t, h, preferred_element_type=jnp.float32)
    cnt_ref[...] += jnp.broadcast_to(
        jnp.sum(oh_t, axis=1, keepdims=True), (B, 128))

    @pl.when(i == NUM_TILES - 1)
    def _finalize():
        counts = cnt_ref[:, 0:1]
        pooled = acc_ref[...] / jnp.maximum(counts, 1.0)
        aux = jnp.dot(pooled, wa_ref[...], preferred_element_type=jnp.float32)
        out_ref[...] = aux + ba_ref[...]


@functools.partial(jax.jit, static_argnames=())
def kernel(entities, batch_index, W_embed, b_embed, W_bb, b_bb, W_aux, b_aux):
    bi = batch_index.astype(jnp.int32).reshape(NUM_TILES, 1, TILE)
    grid = (NUM_TILES,)
    out = pl.pallas_call(
        _fused_kernel,
        grid=grid,
        in_specs=[
            pl.BlockSpec((TILE, D_FEAT), lambda i: (i, 0)),
            pl.BlockSpec((None, 1, TILE), lambda i: (i, 0, 0)),
            pl.BlockSpec((D_FEAT, D_MODEL), lambda i: (0, 0)),
            pl.BlockSpec((1, D_MODEL), lambda i: (0, 0)),
            pl.BlockSpec((D_MODEL, D_MODEL), lambda i: (0, 0)),
            pl.BlockSpec((1, D_MODEL), lambda i: (0, 0)),
            pl.BlockSpec((D_MODEL, AUX_OUT), lambda i: (0, 0)),
            pl.BlockSpec((1, AUX_OUT), lambda i: (0, 0)),
        ],
        out_specs=pl.BlockSpec((B, AUX_OUT), lambda i: (0, 0)),
        out_shape=jax.ShapeDtypeStruct((B, AUX_OUT), jnp.float32),
        scratch_shapes=[
            pltpu.VMEM((B, D_MODEL), jnp.float32),
            pltpu.VMEM((B, 128), jnp.float32),
        ],
    )(entities, bi, W_embed, b_embed.reshape(1, D_MODEL), W_bb,
      b_bb.reshape(1, D_MODEL), W_aux, b_aux.reshape(1, AUX_OUT))
    return out


# speedup vs baseline: 1.2558x; 1.0015x over previous
import functools

import jax
import jax.numpy as jnp
from jax.experimental import pallas as pl
from jax.experimental.pallas import tpu as pltpu

B = 16
TOTAL = 16384
D_FEAT = 256
D_MODEL = 1024
AUX_OUT = 1
TILE = 4096
NUM_TILES = TOTAL // TILE


def _fused_kernel(ent_ref, bi_ref, we_ref, be_ref, wb_ref, bb_ref, wa_ref,
                  ba_ref, out_ref, acc_ref, cnt_ref):
    i = pl.program_id(0)

    @pl.when(i == 0)
    def _init():
        acc_ref[...] = jnp.zeros_like(acc_ref)
        cnt_ref[...] = jnp.zeros_like(cnt_ref)

    x = jnp.dot(ent_ref[...], we_ref[...], preferred_element_type=jnp.float32)
    x = jnp.maximum(x + be_ref[...], 0.0)
    h = jnp.dot(x, wb_ref[...], preferred_element_type=jnp.float32)
    h = jnp.maximum(h + bb_ref[...], 0.0)

    bi = bi_ref[0, :]
    oh_t = (jax.lax.broadcasted_iota(jnp.int32, (B, TILE), 0)
            == bi[None, :]).astype(jnp.float32)
    acc_ref[...] += jnp.dot(oh_t, h, preferred_element_type=jnp.float32)
    cnt_ref[...] += jnp.broadcast_to(
        jnp.sum(oh_t, axis=1, keepdims=True), (B, 128))

    @pl.when(i == NUM_TILES - 1)
    def _finalize():
        counts = cnt_ref[:, 0:1]
        pooled = acc_ref[...] / jnp.maximum(counts, 1.0)
        aux = jnp.dot(pooled, wa_ref[...], preferred_element_type=jnp.float32)
        out_ref[...] = aux + ba_ref[...]


@functools.partial(jax.jit, static_argnames=())
def kernel(entities, batch_index, W_embed, b_embed, W_bb, b_bb, W_aux, b_aux):
    bi = batch_index.astype(jnp.int32).reshape(NUM_TILES, 1, TILE)
    grid = (NUM_TILES,)
    out = pl.pallas_call(
        _fused_kernel,
        grid=grid,
        in_specs=[
            pl.BlockSpec((TILE, D_FEAT), lambda i: (i, 0)),
            pl.BlockSpec((None, 1, TILE), lambda i: (i, 0, 0)),
            pl.BlockSpec((D_FEAT, D_MODEL), lambda i: (0, 0)),
            pl.BlockSpec((1, D_MODEL), lambda i: (0, 0)),
            pl.BlockSpec((D_MODEL, D_MODEL), lambda i: (0, 0)),
            pl.BlockSpec((1, D_MODEL), lambda i: (0, 0)),
            pl.BlockSpec((D_MODEL, AUX_OUT), lambda i: (0, 0)),
            pl.BlockSpec((1, AUX_OUT), lambda i: (0, 0)),
        ],
        out_specs=pl.BlockSpec((B, AUX_OUT), lambda i: (0, 0)),
        out_shape=jax.ShapeDtypeStruct((B, AUX_OUT), jnp.float32),
        scratch_shapes=[
            pltpu.VMEM((B, D_MODEL), jnp.float32),
            pltpu.VMEM((B, 128), jnp.float32),
        ],
    )(entities, bi, W_embed, b_embed.reshape(1, D_MODEL), W_bb,
      b_bb.reshape(1, D_MODEL), W_aux, b_aux.reshape(1, AUX_OUT))
    return out
